# baseline (speedup 1.0000x reference)
import functools

import jax
import jax.numpy as jnp
from jax import lax
from jax.experimental import pallas as pl
from jax.experimental.pallas import tpu as pltpu
from jax.experimental.pallas import tpu_sc as plsc

N_NODES = 6400000
N_GRAPHS = 100000
NB = N_NODES // 128
TC_ROWS = 400
TC_GRID = NB // TC_ROWS
NW = 32
SEG_PER_W = 3200
B_PAD = NW * SEG_PER_W
OFF_PAD = B_PAD + 8
CHUNK = 128
N_CHUNKS = SEG_PER_W // CHUNK


def _tc_body(x_ref, y_ref, z_ref, min_ref,
             gx_ref, gy_ref, gz_ref, p_ref, carry_ref):
    @pl.when(pl.program_id(0) == 0)
    def _init():
        carry_ref[0, 0] = jnp.float32(0.0)

    m = min_ref[...]
    x = x_ref[...]
    y = y_ref[...]
    z = z_ref[...]
    dx = x - m[0, 0]
    dy = y - m[0, 1]
    dz = z - m[0, 2]
    gx_ref[...] = -2.0 * dx
    gy_ref[...] = -2.0 * dy
    gz_ref[...] = -2.0 * dz
    d2 = dx * dx + dy * dy + dz * dz
    li = lax.broadcasted_iota(jnp.int32, (128, 128), 0)
    ci = lax.broadcasted_iota(jnp.int32, (128, 128), 1)
    tu = jnp.where(li <= ci, 1.0, 0.0).astype(jnp.float32)
    w = lax.dot_general(d2, tu, (((1,), (0,)), ((), ())),
                        preferred_element_type=jnp.float32)
    srow = w[:, 127:128]
    rl = lax.broadcasted_iota(jnp.int32, (TC_ROWS, TC_ROWS), 0)
    rc = lax.broadcasted_iota(jnp.int32, (TC_ROWS, TC_ROWS), 1)
    tls = jnp.where(rc < rl, 1.0, 0.0).astype(jnp.float32)
    rp = lax.dot_general(tls, srow, (((1,), (0,)), ((), ())),
                         preferred_element_type=jnp.float32)
    c0 = carry_ref[0, 0]
    p_ref[...] = w + rp + c0
    carry_ref[0, 0] = c0 + jnp.sum(srow)


def _tc_pass(x2, y2, z2, min13):
    blk = pl.BlockSpec((TC_ROWS, 128), lambda i: (i, 0))
    return pl.pallas_call(
        _tc_body,
        grid=(TC_GRID,),
        in_specs=[blk, blk, blk, pl.BlockSpec((1, 3), lambda i: (0, 0))],
        out_specs=[blk, blk, blk, blk],
        out_shape=[
            jax.ShapeDtypeStruct((NB, 128), jnp.float32),
            jax.ShapeDtypeStruct((NB, 128), jnp.float32),
            jax.ShapeDtypeStruct((NB, 128), jnp.float32),
            jax.ShapeDtypeStruct((NB, 128), jnp.float32),
        ],
        scratch_shapes=[pltpu.SMEM((1, 1), jnp.float32)],
        compiler_params=pltpu.CompilerParams(
            dimension_semantics=("arbitrary",)),
    )(x2, y2, z2, min13)


def _sc_body(p_hbm, off_hbm, out_hbm,
             offv, i_e, i_a, m_e, m_a, g_e, g_a, env, sem):
    wid = lax.axis_index("s") * 2 + lax.axis_index("c")
    s0 = wid * SEG_PER_W
    pltpu.sync_copy(off_hbm.at[pl.ds(s0, OFF_PAD - B_PAD + SEG_PER_W)], offv)

    lanes = lax.broadcasted_iota(jnp.int32, (16,), 0)
    zf = jnp.zeros((16,), jnp.float32)
    one = jnp.ones((16,), jnp.float32)

    def index_chunk(k, carry):
        for j in range(CHUNK // 16):
            t = k * CHUNK + j * 16 + lanes
            a = plsc.load_gather(offv, [t])
            b = plsc.load_gather(offv, [t + 1])
            sl = pl.ds(j * 16, 16)
            i_e[k, sl] = jnp.maximum(b - 1, 0)
            i_a[k, sl] = jnp.maximum(a - 1, 0)
            m_e[k, sl] = jnp.where(b > 0, one, zf)
            m_a[k, sl] = jnp.where(a > 0, one, zf)
        pltpu.async_copy(p_hbm.at[i_e.at[k]], g_e.at[k], sem)
        pltpu.async_copy(p_hbm.at[i_a.at[k]], g_a.at[k], sem)
        return carry

    lax.fori_loop(0, N_CHUNKS, index_chunk, 0)

    def drain(k, carry):
        for _ in range(2):
            pltpu.make_async_copy(
                p_hbm.at[pl.ds(0, CHUNK)], g_e.at[k], sem).wait()
        return carry

    lax.fori_loop(0, N_CHUNKS, drain, 0)

    def combine(k, carry):
        for j in range(CHUNK // 16):
            sl = pl.ds(j * 16, 16)
            env[pl.ds(k * CHUNK + j * 16, 16)] = (
                g_e[k, sl] * m_e[k, sl] - g_a[k, sl] * m_a[k, sl])
        return carry

    lax.fori_loop(0, N_CHUNKS, combine, 0)

    pltpu.sync_copy(env, out_hbm.at[pl.ds(s0, SEG_PER_W)])


@functools.cache
def _sc_pass():
    return pl.kernel(
        _sc_body,
        mesh=plsc.VectorSubcoreMesh(core_axis_name="c", subcore_axis_name="s"),
        compiler_params=pltpu.CompilerParams(needs_layout_passes=False),
        out_type=jax.ShapeDtypeStruct((B_PAD,), jnp.float32),
        scratch_types=[
            pltpu.VMEM((OFF_PAD - B_PAD + SEG_PER_W,), jnp.int32),
            pltpu.VMEM((N_CHUNKS, CHUNK), jnp.int32),
            pltpu.VMEM((N_CHUNKS, CHUNK), jnp.int32),
            pltpu.VMEM((N_CHUNKS, CHUNK), jnp.float32),
            pltpu.VMEM((N_CHUNKS, CHUNK), jnp.float32),
            pltpu.VMEM((N_CHUNKS, CHUNK), jnp.float32),
            pltpu.VMEM((N_CHUNKS, CHUNK), jnp.float32),
            pltpu.VMEM((SEG_PER_W,), jnp.float32),
            pltpu.SemaphoreType.DMA,
        ],
    )


def kernel(positions, n_node, minimum):
    x2 = positions[:, 0].reshape(NB, 128)
    y2 = positions[:, 1].reshape(NB, 128)
    z2 = positions[:, 2].reshape(NB, 128)
    gx, gy, gz, p2 = _tc_pass(x2, y2, z2, minimum.reshape(1, 3))

    off_raw = jnp.cumsum(n_node, dtype=jnp.int32)
    off = jnp.minimum(jnp.concatenate(
        [jnp.zeros((1,), jnp.int32), off_raw]), N_NODES)
    off = off.at[N_GRAPHS].set(N_NODES)
    off_pad = jnp.concatenate(
        [off, jnp.full((OFF_PAD - (N_GRAPHS + 1),), N_NODES, jnp.int32)])

    energies_pad = _sc_pass()(p2.reshape(N_NODES), off_pad)
    energies = energies_pad[:N_GRAPHS]

    neg_grad = jnp.stack(
        [gx.reshape(N_NODES), gy.reshape(N_NODES), gz.reshape(N_NODES)],
        axis=1)
    stress = jnp.zeros((6,), positions.dtype)
    return (energies, neg_grad, stress)


# speedup vs baseline: 1.1326x; 1.1326x over previous
import functools

import jax
import jax.numpy as jnp
from jax import lax
from jax.experimental import pallas as pl
from jax.experimental.pallas import tpu as pltpu
from jax.experimental.pallas import tpu_sc as plsc

N_NODES = 6400000
N_GRAPHS = 100000
NB = N_NODES // 128
TC_ROWS = 1000
TC_GRID = NB // TC_ROWS
NW = 32
SEG_PER_W = 3200
B_PAD = NW * SEG_PER_W
OFF_PAD = B_PAD + 8
CHUNK = 128
N_CHUNKS = SEG_PER_W // CHUNK


def _tc_body(x_ref, y_ref, z_ref, min_ref,
             gx_ref, gy_ref, gz_ref, p_ref, carry_ref):
    @pl.when(pl.program_id(0) == 0)
    def _init():
        carry_ref[0, 0] = jnp.float32(0.0)

    m = min_ref[...]
    x = x_ref[...]
    y = y_ref[...]
    z = z_ref[...]
    dx = x - m[0, 0]
    dy = y - m[0, 1]
    dz = z - m[0, 2]
    gx_ref[...] = -2.0 * dx
    gy_ref[...] = -2.0 * dy
    gz_ref[...] = -2.0 * dz
    d2 = dx * dx + dy * dy + dz * dz
    li = lax.broadcasted_iota(jnp.int32, (128, 128), 0)
    ci = lax.broadcasted_iota(jnp.int32, (128, 128), 1)
    tu = jnp.where(li <= ci, 1.0, 0.0).astype(jnp.float32)
    w = lax.dot_general(d2, tu, (((1,), (0,)), ((), ())),
                        preferred_element_type=jnp.float32)
    srow = w[:, 127:128]
    rl = lax.broadcasted_iota(jnp.int32, (TC_ROWS, TC_ROWS), 0)
    rc = lax.broadcasted_iota(jnp.int32, (TC_ROWS, TC_ROWS), 1)
    tls = jnp.where(rc < rl, 1.0, 0.0).astype(jnp.float32)
    rp = lax.dot_general(tls, srow, (((1,), (0,)), ((), ())),
                         preferred_element_type=jnp.float32)
    c0 = carry_ref[0, 0]
    p_ref[...] = w + rp + c0
    carry_ref[0, 0] = c0 + jnp.sum(srow)


def _tc_pass(x2, y2, z2, min13):
    blk = pl.BlockSpec((TC_ROWS, 128), lambda i: (i, 0))
    return pl.pallas_call(
        _tc_body,
        grid=(TC_GRID,),
        in_specs=[blk, blk, blk, pl.BlockSpec((1, 3), lambda i: (0, 0))],
        out_specs=[blk, blk, blk, blk],
        out_shape=[
            jax.ShapeDtypeStruct((NB, 128), jnp.float32),
            jax.ShapeDtypeStruct((NB, 128), jnp.float32),
            jax.ShapeDtypeStruct((NB, 128), jnp.float32),
            jax.ShapeDtypeStruct((NB, 128), jnp.float32),
        ],
        scratch_shapes=[pltpu.SMEM((1, 1), jnp.float32)],
        compiler_params=pltpu.CompilerParams(
            dimension_semantics=("arbitrary",)),
    )(x2, y2, z2, min13)


def _sc_body(p_hbm, off_hbm, out_hbm,
             offv, i_e, i_a, m_e, m_a, g_e, g_a, env, sem):
    wid = lax.axis_index("s") * 2 + lax.axis_index("c")
    s0 = wid * SEG_PER_W
    pltpu.sync_copy(off_hbm.at[pl.ds(s0, OFF_PAD - B_PAD + SEG_PER_W)], offv)

    lanes = lax.broadcasted_iota(jnp.int32, (16,), 0)
    zf = jnp.zeros((16,), jnp.float32)
    one = jnp.ones((16,), jnp.float32)

    def index_chunk(k, carry):
        for j in range(CHUNK // 16):
            t = k * CHUNK + j * 16 + lanes
            a = plsc.load_gather(offv, [t])
            b = plsc.load_gather(offv, [t + 1])
            sl = pl.ds(j * 16, 16)
            i_e[k, sl] = jnp.maximum(b - 1, 0)
            i_a[k, sl] = jnp.maximum(a - 1, 0)
            m_e[k, sl] = jnp.where(b > 0, one, zf)
            m_a[k, sl] = jnp.where(a > 0, one, zf)
        pltpu.async_copy(p_hbm.at[i_e.at[k]], g_e.at[k], sem)
        pltpu.async_copy(p_hbm.at[i_a.at[k]], g_a.at[k], sem)
        return carry

    lax.fori_loop(0, N_CHUNKS, index_chunk, 0)

    def drain(k, carry):
        for _ in range(2):
            pltpu.make_async_copy(
                p_hbm.at[pl.ds(0, CHUNK)], g_e.at[k], sem).wait()
        return carry

    lax.fori_loop(0, N_CHUNKS, drain, 0)

    def combine(k, carry):
        for j in range(CHUNK // 16):
            sl = pl.ds(j * 16, 16)
            env[pl.ds(k * CHUNK + j * 16, 16)] = (
                g_e[k, sl] * m_e[k, sl] - g_a[k, sl] * m_a[k, sl])
        return carry

    lax.fori_loop(0, N_CHUNKS, combine, 0)

    pltpu.sync_copy(env, out_hbm.at[pl.ds(s0, SEG_PER_W)])


@functools.cache
def _sc_pass():
    return pl.kernel(
        _sc_body,
        mesh=plsc.VectorSubcoreMesh(core_axis_name="c", subcore_axis_name="s"),
        compiler_params=pltpu.CompilerParams(needs_layout_passes=False),
        out_type=jax.ShapeDtypeStruct((B_PAD,), jnp.float32),
        scratch_types=[
            pltpu.VMEM((OFF_PAD - B_PAD + SEG_PER_W,), jnp.int32),
            pltpu.VMEM((N_CHUNKS, CHUNK), jnp.int32),
            pltpu.VMEM((N_CHUNKS, CHUNK), jnp.int32),
            pltpu.VMEM((N_CHUNKS, CHUNK), jnp.float32),
            pltpu.VMEM((N_CHUNKS, CHUNK), jnp.float32),
            pltpu.VMEM((N_CHUNKS, CHUNK), jnp.float32),
            pltpu.VMEM((N_CHUNKS, CHUNK), jnp.float32),
            pltpu.VMEM((SEG_PER_W,), jnp.float32),
            pltpu.SemaphoreType.DMA,
        ],
    )


def kernel(positions, n_node, minimum):
    x2 = positions[:, 0].reshape(NB, 128)
    y2 = positions[:, 1].reshape(NB, 128)
    z2 = positions[:, 2].reshape(NB, 128)
    gx, gy, gz, p2 = _tc_pass(x2, y2, z2, minimum.reshape(1, 3))

    off_raw = jnp.cumsum(n_node, dtype=jnp.int32)
    off = jnp.minimum(jnp.concatenate(
        [jnp.zeros((1,), jnp.int32), off_raw]), N_NODES)
    off = off.at[N_GRAPHS].set(N_NODES)
    off_pad = jnp.concatenate(
        [off, jnp.full((OFF_PAD - (N_GRAPHS + 1),), N_NODES, jnp.int32)])

    energies_pad = _sc_pass()(p2.reshape(N_NODES), off_pad)
    energies = energies_pad[:N_GRAPHS]

    neg_grad = jnp.stack(
        [gx.reshape(N_NODES), gy.reshape(N_NODES), gz.reshape(N_NODES)],
        axis=1)
    stress = jnp.zeros((6,), positions.dtype)
    return (energies, neg_grad, stress)


# speedup vs baseline: 1.1729x; 1.0356x over previous
import functools

import jax
import jax.numpy as jnp
from jax import lax
from jax.experimental import pallas as pl
from jax.experimental.pallas import tpu as pltpu
from jax.experimental.pallas import tpu_sc as plsc

N_NODES = 6400000
N_GRAPHS = 100000
NB = N_NODES // 128
TC_ROWS = 2000
TC_GRID = NB // TC_ROWS
NW = 32
SEG_PER_W = 3200
B_PAD = NW * SEG_PER_W
OFF_PAD = B_PAD + 8
CHUNK = 128
N_CHUNKS = SEG_PER_W // CHUNK


def _tc_body(x_ref, y_ref, z_ref, min_ref,
             gx_ref, gy_ref, gz_ref, p_ref, carry_ref):
    @pl.when(pl.program_id(0) == 0)
    def _init():
        carry_ref[0, 0] = jnp.float32(0.0)

    m = min_ref[...]
    x = x_ref[...]
    y = y_ref[...]
    z = z_ref[...]
    dx = x - m[0, 0]
    dy = y - m[0, 1]
    dz = z - m[0, 2]
    gx_ref[...] = -2.0 * dx
    gy_ref[...] = -2.0 * dy
    gz_ref[...] = -2.0 * dz
    d2 = dx * dx + dy * dy + dz * dz
    li = lax.broadcasted_iota(jnp.int32, (128, 128), 0)
    ci = lax.broadcasted_iota(jnp.int32, (128, 128), 1)
    tu = jnp.where(li <= ci, 1.0, 0.0).astype(jnp.float32)
    w = lax.dot_general(d2, tu, (((1,), (0,)), ((), ())),
                        preferred_element_type=jnp.float32)
    srow = w[:, 127:128]
    rl = lax.broadcasted_iota(jnp.int32, (TC_ROWS, TC_ROWS), 0)
    rc = lax.broadcasted_iota(jnp.int32, (TC_ROWS, TC_ROWS), 1)
    tls = jnp.where(rc < rl, 1.0, 0.0).astype(jnp.float32)
    rp = lax.dot_general(tls, srow, (((1,), (0,)), ((), ())),
                         preferred_element_type=jnp.float32)
    c0 = carry_ref[0, 0]
    p_ref[...] = w + rp + c0
    carry_ref[0, 0] = c0 + jnp.sum(srow)


def _tc_pass(x2, y2, z2, min13):
    blk = pl.BlockSpec((TC_ROWS, 128), lambda i: (i, 0))
    return pl.pallas_call(
        _tc_body,
        grid=(TC_GRID,),
        in_specs=[blk, blk, blk, pl.BlockSpec((1, 3), lambda i: (0, 0))],
        out_specs=[blk, blk, blk, blk],
        out_shape=[
            jax.ShapeDtypeStruct((NB, 128), jnp.float32),
            jax.ShapeDtypeStruct((NB, 128), jnp.float32),
            jax.ShapeDtypeStruct((NB, 128), jnp.float32),
            jax.ShapeDtypeStruct((NB, 128), jnp.float32),
        ],
        scratch_shapes=[pltpu.SMEM((1, 1), jnp.float32)],
        compiler_params=pltpu.CompilerParams(
            dimension_semantics=("arbitrary",)),
    )(x2, y2, z2, min13)


def _sc_body(p_hbm, off_hbm, out_hbm,
             offv, i_e, i_a, m_e, m_a, g_e, g_a, env, sem):
    wid = lax.axis_index("s") * 2 + lax.axis_index("c")
    s0 = wid * SEG_PER_W
    pltpu.sync_copy(off_hbm.at[pl.ds(s0, OFF_PAD - B_PAD + SEG_PER_W)], offv)

    lanes = lax.broadcasted_iota(jnp.int32, (16,), 0)
    zf = jnp.zeros((16,), jnp.float32)
    one = jnp.ones((16,), jnp.float32)

    def index_chunk(k, carry):
        for j in range(CHUNK // 16):
            t = k * CHUNK + j * 16 + lanes
            a = plsc.load_gather(offv, [t])
            b = plsc.load_gather(offv, [t + 1])
            sl = pl.ds(j * 16, 16)
            i_e[k, sl] = jnp.maximum(b - 1, 0)
            i_a[k, sl] = jnp.maximum(a - 1, 0)
            m_e[k, sl] = jnp.where(b > 0, one, zf)
            m_a[k, sl] = jnp.where(a > 0, one, zf)
        pltpu.async_copy(p_hbm.at[i_e.at[k]], g_e.at[k], sem)
        pltpu.async_copy(p_hbm.at[i_a.at[k]], g_a.at[k], sem)
        return carry

    lax.fori_loop(0, N_CHUNKS, index_chunk, 0)

    def drain(k, carry):
        for _ in range(2):
            pltpu.make_async_copy(
                p_hbm.at[pl.ds(0, CHUNK)], g_e.at[k], sem).wait()
        return carry

    lax.fori_loop(0, N_CHUNKS, drain, 0)

    def combine(k, carry):
        for j in range(CHUNK // 16):
            sl = pl.ds(j * 16, 16)
            env[pl.ds(k * CHUNK + j * 16, 16)] = (
                g_e[k, sl] * m_e[k, sl] - g_a[k, sl] * m_a[k, sl])
        return carry

    lax.fori_loop(0, N_CHUNKS, combine, 0)

    pltpu.sync_copy(env, out_hbm.at[pl.ds(s0, SEG_PER_W)])


@functools.cache
def _sc_pass():
    return pl.kernel(
        _sc_body,
        mesh=plsc.VectorSubcoreMesh(core_axis_name="c", subcore_axis_name="s"),
        compiler_params=pltpu.CompilerParams(needs_layout_passes=False),
        out_type=jax.ShapeDtypeStruct((B_PAD,), jnp.float32),
        scratch_types=[
            pltpu.VMEM((OFF_PAD - B_PAD + SEG_PER_W,), jnp.int32),
            pltpu.VMEM((N_CHUNKS, CHUNK), jnp.int32),
            pltpu.VMEM((N_CHUNKS, CHUNK), jnp.int32),
            pltpu.VMEM((N_CHUNKS, CHUNK), jnp.float32),
            pltpu.VMEM((N_CHUNKS, CHUNK), jnp.float32),
            pltpu.VMEM((N_CHUNKS, CHUNK), jnp.float32),
            pltpu.VMEM((N_CHUNKS, CHUNK), jnp.float32),
            pltpu.VMEM((SEG_PER_W,), jnp.float32),
            pltpu.SemaphoreType.DMA,
        ],
    )


def kernel(positions, n_node, minimum):
    x2 = positions[:, 0].reshape(NB, 128)
    y2 = positions[:, 1].reshape(NB, 128)
    z2 = positions[:, 2].reshape(NB, 128)
    gx, gy, gz, p2 = _tc_pass(x2, y2, z2, minimum.reshape(1, 3))

    off_raw = jnp.cumsum(n_node, dtype=jnp.int32)
    off = jnp.minimum(jnp.concatenate(
        [jnp.zeros((1,), jnp.int32), off_raw]), N_NODES)
    off = off.at[N_GRAPHS].set(N_NODES)
    off_pad = jnp.concatenate(
        [off, jnp.full((OFF_PAD - (N_GRAPHS + 1),), N_NODES, jnp.int32)])

    energies_pad = _sc_pass()(p2.reshape(N_NODES), off_pad)
    energies = energies_pad[:N_GRAPHS]

    neg_grad = jnp.stack(
        [gx.reshape(N_NODES), gy.reshape(N_NODES), gz.reshape(N_NODES)],
        axis=1)
    stress = jnp.zeros((6,), positions.dtype)
    return (energies, neg_grad, stress)
